# Initial kernel scaffold; baseline (speedup 1.0000x reference)
import jax
import jax.numpy as jnp
from jax.experimental import pallas as pl


def kernel(inputs, embeddings):
    raise NotImplementedError("write your pallas kernel here")


# speedup vs baseline: 1.8205x; 1.8205x over previous
import functools

import jax
import jax.numpy as jnp
from jax import lax
from jax.experimental import pallas as pl
from jax.experimental.pallas import tpu as pltpu
from jax.experimental.pallas import tpu_sc as plsc

_NC = 2
_NS = 16
_NW = _NC * _NS
_IW = 128
_G = 4
_CHUNK = _G * _IW


@functools.partial(jax.jit, static_argnums=(2, 3))
def _gather(table, idx2d, n_rows, d):
    rows_per_w = n_rows // _NW
    chunks = rows_per_w // _G
    b_per_w = rows_per_w * _IW

    mesh = plsc.VectorSubcoreMesh(core_axis_name="c", subcore_axis_name="s")

    @functools.partial(
        pl.kernel,
        out_type=jax.ShapeDtypeStruct((n_rows * _IW, d), jnp.float32),
        mesh=mesh,
        compiler_params=pltpu.CompilerParams(use_tc_tiling_on_sc=False),
        scratch_types=[
            pltpu.VMEM((rows_per_w, _IW), jnp.int32),
            pltpu.VMEM((_CHUNK, d), jnp.float32),
            pltpu.SemaphoreType.DMA,
        ],
    )
    def k(table_hbm, idx_hbm, out_hbm, idx_v, rows_v, sem):
        wid = lax.axis_index("s") * _NC + lax.axis_index("c")
        pltpu.sync_copy(idx_hbm.at[pl.ds(wid * rows_per_w, rows_per_w)], idx_v)
        out_base = wid * b_per_w

        def body(c, carry):
            copies = [
                pltpu.async_copy(
                    table_hbm.at[idx_v.at[c * _G + g]],
                    rows_v.at[pl.ds(g * _IW, _IW)],
                    sem,
                )
                for g in range(_G)
            ]
            for cp in copies:
                cp.wait()
            pltpu.sync_copy(
                rows_v, out_hbm.at[pl.ds(out_base + c * _CHUNK, _CHUNK)]
            )
            return carry

        lax.fori_loop(0, chunks, body, 0)

    return k(table, idx2d)


def kernel(inputs, embeddings):
    b, h = inputs.shape
    d = embeddings.shape[-1]
    flat = inputs.reshape(-1).astype(jnp.int32)
    n = flat.shape[0]
    idx2d = flat.reshape(n // _IW, _IW)
    out = _gather(embeddings, idx2d, n // _IW, d)
    return out.reshape(b, h, d)


# speedup vs baseline: 1.8646x; 1.0242x over previous
import functools

import jax
import jax.numpy as jnp
from jax import lax
from jax.experimental import pallas as pl
from jax.experimental.pallas import tpu as pltpu
from jax.experimental.pallas import tpu_sc as plsc

_NC = 2
_NS = 16
_NW = _NC * _NS
_IW = 128
_G = 4
_CHUNK = _G * _IW


@functools.partial(jax.jit, static_argnums=(2, 3))
def _gather(table, idx2d, n_rows, d):
    rows_per_w = n_rows // _NW
    chunks = rows_per_w // _G
    b_per_w = rows_per_w * _IW

    mesh = plsc.VectorSubcoreMesh(core_axis_name="c", subcore_axis_name="s")

    @functools.partial(
        pl.kernel,
        out_type=jax.ShapeDtypeStruct((n_rows * _IW, d), jnp.float32),
        mesh=mesh,
        compiler_params=pltpu.CompilerParams(use_tc_tiling_on_sc=False),
        scratch_types=[
            pltpu.VMEM((rows_per_w, _IW), jnp.int32),
            pltpu.VMEM((_CHUNK, d), jnp.float32),
            pltpu.VMEM((_CHUNK, d), jnp.float32),
            pltpu.SemaphoreType.DMA,
            pltpu.SemaphoreType.DMA,
            pltpu.SemaphoreType.DMA,
            pltpu.SemaphoreType.DMA,
        ],
    )
    def k(table_hbm, idx_hbm, out_hbm, idx_v, rows0, rows1, g0, g1, s0, s1):
        wid = lax.axis_index("s") * _NC + lax.axis_index("c")
        pltpu.sync_copy(idx_hbm.at[pl.ds(wid * rows_per_w, rows_per_w)], idx_v)
        out_base = wid * b_per_w
        bufs = ((rows0, g0, s0), (rows1, g1, s1))

        def fire(c, rbuf, gsem):
            for g in range(_G):
                pltpu.async_copy(
                    table_hbm.at[idx_v.at[c * _G + g]],
                    rbuf.at[pl.ds(g * _IW, _IW)],
                    gsem,
                )

        def wait_gathers(rbuf, gsem):
            for g in range(_G):
                pltpu.make_async_copy(
                    table_hbm.at[idx_v.at[0]],
                    rbuf.at[pl.ds(g * _IW, _IW)],
                    gsem,
                ).wait()

        def store(c, rbuf, ssem):
            pltpu.async_copy(
                rbuf, out_hbm.at[pl.ds(out_base + c * _CHUNK, _CHUNK)], ssem
            )

        def wait_store(rbuf, ssem):
            pltpu.make_async_copy(
                rbuf, out_hbm.at[pl.ds(out_base, _CHUNK)], ssem
            ).wait()

        fire(0, rows0, g0)
        fire(1, rows1, g1)

        def pair(i, carry):
            for b, (rbuf, gsem, ssem) in enumerate(bufs):
                c = 2 * i + b
                wait_gathers(rbuf, gsem)
                store(c, rbuf, ssem)
                wait_store(rbuf, ssem)
                fire(c + 2, rbuf, gsem)
            return carry

        lax.fori_loop(0, chunks // 2 - 1, pair, 0)

        for b, (rbuf, gsem, ssem) in enumerate(bufs):
            wait_gathers(rbuf, gsem)
            store(chunks - 2 + b, rbuf, ssem)
            wait_store(rbuf, ssem)

    return k(table, idx2d)


def kernel(inputs, embeddings):
    b, h = inputs.shape
    d = embeddings.shape[-1]
    flat = inputs.reshape(-1).astype(jnp.int32)
    n = flat.shape[0]
    idx2d = flat.reshape(n // _IW, _IW)
    out = _gather(embeddings, idx2d, n // _IW, d)
    return out.reshape(b, h, d)


# speedup vs baseline: 1.8649x; 1.0001x over previous
import functools

import jax
import jax.numpy as jnp
from jax import lax
from jax.experimental import pallas as pl
from jax.experimental.pallas import tpu as pltpu
from jax.experimental.pallas import tpu_sc as plsc

_NC = 2
_NS = 16
_NW = _NC * _NS
_IW = 128
_G = 4
_CHUNK = _G * _IW


@functools.partial(jax.jit, static_argnums=(2, 3))
def _gather(table, idx, n, d):
    b_per_w = n // _NW
    chunks = b_per_w // _CHUNK

    mesh = plsc.VectorSubcoreMesh(core_axis_name="c", subcore_axis_name="s")

    @functools.partial(
        pl.kernel,
        out_type=jax.ShapeDtypeStruct((n, d), jnp.float32),
        mesh=mesh,
        compiler_params=pltpu.CompilerParams(use_tc_tiling_on_sc=False),
        scratch_types=[
            pltpu.VMEM((b_per_w,), jnp.int32),
            pltpu.VMEM((_CHUNK, d), jnp.float32),
            pltpu.VMEM((_CHUNK, d), jnp.float32),
            pltpu.SemaphoreType.DMA,
            pltpu.SemaphoreType.DMA,
            pltpu.SemaphoreType.DMA,
            pltpu.SemaphoreType.DMA,
        ],
    )
    def k(table_hbm, idx_hbm, out_hbm, idx_v, rows0, rows1, g0, g1, s0, s1):
        wid = lax.axis_index("s") * _NC + lax.axis_index("c")
        pltpu.sync_copy(idx_hbm.at[pl.ds(wid * b_per_w, b_per_w)], idx_v)
        out_base = wid * b_per_w
        bufs = ((rows0, g0, s0), (rows1, g1, s1))

        def fire(c, rbuf, gsem):
            pltpu.async_copy(
                table_hbm.at[idx_v.at[pl.ds(c * _CHUNK, _CHUNK)]],
                rbuf,
                gsem,
            )

        def wait_gathers(rbuf, gsem):
            pltpu.make_async_copy(
                table_hbm.at[idx_v.at[pl.ds(0, _CHUNK)]],
                rbuf,
                gsem,
            ).wait()

        def store(c, rbuf, ssem):
            pltpu.async_copy(
                rbuf, out_hbm.at[pl.ds(out_base + c * _CHUNK, _CHUNK)], ssem
            )

        def wait_store(rbuf, ssem):
            pltpu.make_async_copy(
                rbuf, out_hbm.at[pl.ds(out_base, _CHUNK)], ssem
            ).wait()

        fire(0, rows0, g0)
        fire(1, rows1, g1)

        def pair(i, carry):
            for b, (rbuf, gsem, ssem) in enumerate(bufs):
                c = 2 * i + b
                wait_gathers(rbuf, gsem)
                store(c, rbuf, ssem)
                wait_store(rbuf, ssem)
                fire(c + 2, rbuf, gsem)
            return carry

        lax.fori_loop(0, chunks // 2 - 1, pair, 0)

        for b, (rbuf, gsem, ssem) in enumerate(bufs):
            wait_gathers(rbuf, gsem)
            store(chunks - 2 + b, rbuf, ssem)
            wait_store(rbuf, ssem)

    return k(table, idx)


def kernel(inputs, embeddings):
    b, h = inputs.shape
    d = embeddings.shape[-1]
    flat = inputs.reshape(-1).astype(jnp.int32)
    out = _gather(embeddings, flat, flat.shape[0], d)
    return out.reshape(b, h, d)
